# baseline (speedup 1.0000x reference)
import functools

import jax
import jax.numpy as jnp
from jax.experimental import pallas as pl
from jax.experimental.pallas import tpu as pltpu

_CHUNK = 16


def _bilstm_kernel(xf_ref, xb_ref, mf_ref, mb_ref, wih_ref, whh_ref, b_ref,
                   of_ref, ob_ref,
                   hf_ref, cf_ref, hb_ref, cb_ref,
                   xgf_ref, xgb_ref, *, chunk, hd):
    cstep = pl.program_id(1)
    B2 = xf_ref.shape[1]
    D = xf_ref.shape[2]

    @pl.when(cstep == 0)
    def _init():
        hf_ref[...] = jnp.zeros_like(hf_ref)
        cf_ref[...] = jnp.zeros_like(cf_ref)
        hb_ref[...] = jnp.zeros_like(hb_ref)
        cb_ref[...] = jnp.zeros_like(cb_ref)

    def project(x_ref, w_idx, xg_w):
        xg = jnp.dot(x_ref[...].reshape(chunk * B2, D), wih_ref[w_idx],
                     preferred_element_type=jnp.float32)
        xg_w[...] = (xg + b_ref[w_idx]).astype(jnp.bfloat16)

    project(xf_ref, 0, xgf_ref)
    project(xb_ref, 1, xgb_ref)

    def gates(g, c_prev):
        i_g = jax.nn.sigmoid(g[:, 0 * hd:1 * hd])
        f_g = jax.nn.sigmoid(g[:, 1 * hd:2 * hd])
        g_g = jnp.tanh(g[:, 2 * hd:3 * hd])
        o_g = jax.nn.sigmoid(g[:, 3 * hd:4 * hd])
        c_new = f_g * c_prev + i_g * g_g
        h_new = o_g * jnp.tanh(c_new)
        return h_new, c_new

    h_f, c_f = hf_ref[...], cf_ref[...]
    h_b, c_b = hb_ref[...], cb_ref[...]
    for j in range(chunk):
        tb = chunk - 1 - j
        g_f = (xgf_ref[pl.ds(j * B2, B2), :].astype(jnp.float32)
               + jnp.dot(h_f.astype(jnp.bfloat16), whh_ref[0],
                         preferred_element_type=jnp.float32))
        g_b = (xgb_ref[pl.ds(tb * B2, B2), :].astype(jnp.float32)
               + jnp.dot(h_b.astype(jnp.bfloat16), whh_ref[1],
                         preferred_element_type=jnp.float32))
        hn_f, cn_f = gates(g_f, c_f)
        hn_b, cn_b = gates(g_b, c_b)
        m_f = mf_ref[j]
        m_b = mb_ref[tb]
        h_f = h_f + m_f * (hn_f - h_f)
        c_f = c_f + m_f * (cn_f - c_f)
        h_b = h_b + m_b * (hn_b - h_b)
        c_b = c_b + m_b * (cn_b - c_b)
        of_ref[j] = m_f * hn_f
        ob_ref[tb] = m_b * hn_b
    hf_ref[...], cf_ref[...] = h_f, c_f
    hb_ref[...], cb_ref[...] = h_b, c_b


def kernel(inputs, mask, w_ih_f, w_hh_f, b_ih_f, b_hh_f,
           w_ih_b, w_hh_b, b_ih_b, b_hh_b):
    B, S, D = inputs.shape
    hd = w_hh_f.shape[1]
    out_dtype = inputs.dtype
    C = _CHUNK if S % _CHUNK == 0 else S
    NC = S // C
    B2 = B // 2 if B % 2 == 0 else B
    GB = B // B2

    x_t = jnp.transpose(inputs, (1, 0, 2)).astype(jnp.bfloat16)
    m_t = jnp.transpose(mask.astype(jnp.float32), (1, 0))[:, :, None]
    wih = jnp.stack([jnp.transpose(w_ih_f), jnp.transpose(w_ih_b)]
                    ).astype(jnp.bfloat16)
    whh = jnp.stack([jnp.transpose(w_hh_f), jnp.transpose(w_hh_b)]
                    ).astype(jnp.bfloat16)
    bias = jnp.stack([b_ih_f + b_hh_f, b_ih_b + b_hh_b]
                     ).astype(jnp.float32)[:, None, :]

    body = functools.partial(_bilstm_kernel, chunk=C, hd=hd)

    out_f, out_b = pl.pallas_call(
        body,
        out_shape=(jax.ShapeDtypeStruct((S, B, hd), jnp.float32),
                   jax.ShapeDtypeStruct((S, B, hd), jnp.float32)),
        grid_spec=pltpu.PrefetchScalarGridSpec(
            num_scalar_prefetch=0,
            grid=(GB, NC),
            in_specs=[
                pl.BlockSpec((C, B2, D), lambda bh, c: (c, bh, 0)),
                pl.BlockSpec((C, B2, D), lambda bh, c: (NC - 1 - c, bh, 0)),
                pl.BlockSpec((C, B2, 1), lambda bh, c: (c, bh, 0)),
                pl.BlockSpec((C, B2, 1), lambda bh, c: (NC - 1 - c, bh, 0)),
                pl.BlockSpec((2, D, 4 * hd), lambda bh, c: (0, 0, 0)),
                pl.BlockSpec((2, hd, 4 * hd), lambda bh, c: (0, 0, 0)),
                pl.BlockSpec((2, 1, 4 * hd), lambda bh, c: (0, 0, 0)),
            ],
            out_specs=(
                pl.BlockSpec((C, B2, hd), lambda bh, c: (c, bh, 0)),
                pl.BlockSpec((C, B2, hd), lambda bh, c: (NC - 1 - c, bh, 0)),
            ),
            scratch_shapes=[
                pltpu.VMEM((B2, hd), jnp.float32),
                pltpu.VMEM((B2, hd), jnp.float32),
                pltpu.VMEM((B2, hd), jnp.float32),
                pltpu.VMEM((B2, hd), jnp.float32),
                pltpu.VMEM((C * B2, 4 * hd), jnp.bfloat16),
                pltpu.VMEM((C * B2, 4 * hd), jnp.bfloat16),
            ],
        ),
        compiler_params=pltpu.CompilerParams(
            dimension_semantics=("parallel", "arbitrary")),
    )(x_t, x_t, m_t, m_t, wih, whh, bias)

    out_t = jnp.concatenate([out_f, out_b], axis=-1)
    return jnp.transpose(out_t, (1, 0, 2)).astype(out_dtype)
